# baseline (device time: 19240 ns/iter reference)
import jax
import jax.numpy as jnp
from jax import lax
from jax.experimental import pallas as pl
from jax.experimental.pallas import tpu as pltpu

N_DEV = 4
DH = 64


def kernel(x, Wq, Wo, Wk, Wv):
    B, Sq, D = x.shape
    d_sh = Wq.shape[1]
    H_sh = d_sh // DH
    R = B * Sq
    NH = B * H_sh
    BR = R // N_DEV

    def body(x_ref, wq_ref, wo_ref, wk_ref, wv_ref, out_ref,
             s_ref, send_ref, recv_ref, send_sems, recv_sems):
        my = lax.axis_index("i")

        xf = x_ref[...].reshape(R, D)
        q = jnp.dot(xf, wq_ref[...], preferred_element_type=jnp.float32)
        k = jnp.dot(xf, wk_ref[...], preferred_element_type=jnp.float32)
        v = jnp.dot(xf, wv_ref[...], preferred_element_type=jnp.float32)

        for b in range(B):
            rows = slice(b * Sq, (b + 1) * Sq)
            for h in range(H_sh):
                cols = slice(h * DH, (h + 1) * DH)
                i = b * H_sh + h
                s_ref[i * Sq:(i + 1) * Sq, :] = jnp.dot(
                    q[rows, cols], k[rows, cols].T,
                    preferred_element_type=jnp.float32,
                ) * 0.125

        p = jnp.exp(s_ref[...])
        p = p / jnp.sum(p, axis=-1, keepdims=True)

        barrier_sem = pltpu.get_barrier_semaphore()
        for j in range(1, N_DEV):
            pl.semaphore_signal(
                barrier_sem, inc=1,
                device_id=(lax.rem(my + j, N_DEV),),
                device_id_type=pl.DeviceIdType.MESH,
            )
        pl.semaphore_wait(barrier_sem, N_DEV - 1)

        sends = []
        own = []
        for bk in range(N_DEV):
            b = bk // 2
            r0 = (bk % 2) * BR
            outs = []
            for h in range(H_sh):
                cols = slice(h * DH, (h + 1) * DH)
                i = b * H_sh + h
                outs.append(jnp.dot(
                    p[i * Sq + r0:i * Sq + r0 + BR, :],
                    v[b * Sq:(b + 1) * Sq, cols],
                    preferred_element_type=jnp.float32,
                ))
            att_blk = jnp.concatenate(outs, axis=1)
            blk = jnp.dot(
                att_blk, wo_ref[...], preferred_element_type=jnp.float32
            )
            own.append(blk)
            send_ref[bk] = blk.astype(jnp.bfloat16)
            for j in range(1, N_DEV):
                rd = pltpu.make_async_remote_copy(
                    src_ref=send_ref.at[bk],
                    dst_ref=recv_ref.at[3 - j, bk],
                    send_sem=send_sems.at[j - 1, bk],
                    recv_sem=recv_sems.at[3 - j, bk],
                    device_id=(lax.rem(my + j, N_DEV),),
                    device_id_type=pl.DeviceIdType.MESH,
                )
                rd.start()
                sends.append(rd)

        for bk in range(N_DEV):
            b = bk // 2
            orows = pl.ds((bk % 2) * BR, BR)
            for s in range(N_DEV - 1):
                rwait = pltpu.make_async_remote_copy(
                    src_ref=send_ref.at[0],
                    dst_ref=recv_ref.at[s, bk],
                    send_sem=send_sems.at[0, 0],
                    recv_sem=recv_sems.at[s, bk],
                    device_id=(my,),
                    device_id_type=pl.DeviceIdType.MESH,
                )
                rwait.wait_recv()
            out_ref[b, orows, :] = (
                own[bk]
                + recv_ref[0, bk].astype(jnp.float32)
                + recv_ref[1, bk].astype(jnp.float32)
                + recv_ref[2, bk].astype(jnp.float32)
            )

        for rd in sends:
            rd.wait_send()

    return pl.pallas_call(
        body,
        out_shape=jax.ShapeDtypeStruct((B, Sq, D), jnp.float32),
        in_specs=[pl.BlockSpec(memory_space=pltpu.VMEM)] * 5,
        out_specs=pl.BlockSpec(memory_space=pltpu.VMEM),
        scratch_shapes=[
            pltpu.VMEM((NH * Sq, Sq), jnp.float32),
            pltpu.VMEM((N_DEV, BR, D), jnp.bfloat16),
            pltpu.VMEM((N_DEV - 1, N_DEV, BR, D), jnp.bfloat16),
            pltpu.SemaphoreType.DMA((N_DEV - 1, N_DEV)),
            pltpu.SemaphoreType.DMA((N_DEV - 1, N_DEV)),
        ],
        compiler_params=pltpu.CompilerParams(collective_id=0),
    )(x, Wq, Wo, Wk, Wv)
